# baseline (speedup 1.0000x reference)
import functools

import jax
import jax.numpy as jnp
from jax import lax
from jax.experimental import pallas as pl
from jax.experimental.pallas import tpu as pltpu
from jax.experimental.pallas import tpu_sc as plsc

B = 16384
N = 1000000
D = 64
NC = 2
NS = 16
NW = NC * NS
BPW = B // NW
L = 16
NG = BPW // L
CH = 256
NCH = BPW // CH
CHG = CH // L
IW = 128


def _sc_body(uidx_hbm, pidx_hbm, uflat_hbm, pflat_hbm, ubias_hbm, pbias_hbm,
             out_hbm, dummy_hbm, uidx_v, pidx_v, iu_v, ip_v, ucols_v, pcols_v,
             ub_v, pb_v, out_v, sem, sem2):
    c = lax.axis_index("c")
    s = lax.axis_index("s")
    wid = s * NC + c
    base = wid * BPW

    pltpu.sync_copy(uidx_hbm.at[pl.ds(base, BPW)], uidx_v)
    pltpu.sync_copy(pidx_hbm.at[pl.ds(base, BPW)], pidx_v)

    pltpu.async_copy(ubias_hbm.at[uidx_v], ub_v, sem2)
    pltpu.async_copy(pbias_hbm.at[pidx_v], pb_v, sem2)

    def build(g, carry):
        sl = pl.ds(g * L, L)
        iu = uidx_v[sl]
        ip = pidx_v[sl]

        def bd(d, carry):
            iu_v[d, sl] = iu + d * N
            ip_v[d, sl] = ip + d * N
            return carry

        lax.fori_loop(0, D, bd, 0)
        return carry

    lax.fori_loop(0, NG, build, 0)

    def chunk(cc, carry):
        cbase = cc * CH

        def issue(d, carry):
            for h in range(CH // IW):
                o = cbase + h * IW
                pltpu.async_copy(uflat_hbm.at[iu_v.at[d, pl.ds(o, IW)]],
                                 ucols_v.at[d, pl.ds(h * IW, IW)], sem)
                pltpu.async_copy(pflat_hbm.at[ip_v.at[d, pl.ds(o, IW)]],
                                 pcols_v.at[d, pl.ds(h * IW, IW)], sem)
            return carry

        lax.fori_loop(0, D, issue, 0)
        pltpu.make_async_copy(dummy_hbm, ucols_v, sem).wait()
        pltpu.make_async_copy(dummy_hbm, pcols_v, sem).wait()

        def body(g, carry):
            sl = pl.ds(g * L, L)
            acc = ucols_v[0, sl] * pcols_v[0, sl]
            for d in range(1, D):
                acc = acc + ucols_v[d, sl] * pcols_v[d, sl]
            out_v[pl.ds(cbase + g * L, L)] = acc
            return carry

        lax.fori_loop(0, CHG, body, 0)
        return carry

    lax.fori_loop(0, NCH, chunk, 0)

    pltpu.make_async_copy(ubias_hbm.at[pl.ds(0, BPW)], ub_v, sem2).wait()
    pltpu.make_async_copy(pbias_hbm.at[pl.ds(0, BPW)], pb_v, sem2).wait()

    def bias_body(g, carry):
        sl = pl.ds(g * L, L)
        out_v[sl] = out_v[sl] + ub_v[sl] + pb_v[sl]
        return carry

    lax.fori_loop(0, NG, bias_body, 0)
    pltpu.sync_copy(out_v, out_hbm.at[pl.ds(base, BPW)])


@jax.jit
def _run(uidx, pidx, uflat, pflat, ubias, pbias):
    mesh = plsc.VectorSubcoreMesh(core_axis_name="c", subcore_axis_name="s")
    kern = functools.partial(
        pl.kernel,
        out_type=(jax.ShapeDtypeStruct((B,), jnp.float32),
                  jax.ShapeDtypeStruct((D, CH), jnp.float32)),
        mesh=mesh,
        compiler_params=pltpu.CompilerParams(
            needs_layout_passes=False, use_tc_tiling_on_sc=False),
        scratch_types=[
            pltpu.VMEM((BPW,), jnp.int32),
            pltpu.VMEM((BPW,), jnp.int32),
            pltpu.VMEM((D, BPW), jnp.int32),
            pltpu.VMEM((D, BPW), jnp.int32),
            pltpu.VMEM((D, CH), jnp.float32),
            pltpu.VMEM((D, CH), jnp.float32),
            pltpu.VMEM((BPW,), jnp.float32),
            pltpu.VMEM((BPW,), jnp.float32),
            pltpu.VMEM((BPW,), jnp.float32),
            pltpu.SemaphoreType.DMA,
            pltpu.SemaphoreType.DMA,
        ],
    )(_sc_body)
    out, _ = kern(uidx, pidx, uflat, pflat, ubias, pbias)
    return out


def kernel(inputs, user_emb, place_emb, user_bias, place_bias):
    uidx = inputs[:, 0]
    pidx = inputs[:, 1]
    out = _run(uidx, pidx,
               user_emb.T.reshape(-1), place_emb.T.reshape(-1),
               user_bias.reshape(-1), place_bias.reshape(-1))
    return out.reshape(B, 1)


# speedup vs baseline: 9.1394x; 9.1394x over previous
import functools

import jax
import jax.numpy as jnp
from jax import lax
from jax.experimental import pallas as pl
from jax.experimental.pallas import tpu as pltpu
from jax.experimental.pallas import tpu_sc as plsc

B = 16384
N = 1000000
D = 64
NC = 2
NS = 16
NW = NC * NS
BPW = B // NW
L = 16
CH = 256
NCH = BPW // CH
CHG = CH // L
PW = 128


def _sc_body(uidx_hbm, pidx_hbm, uq_hbm, pq_hbm, ubias_hbm, pbias_hbm,
             out_hbm, uidx_v, pidx_v, urows_v, prows_v, ub_v, pb_v, out_v,
             sem, sem2):
    c = lax.axis_index("c")
    s = lax.axis_index("s")
    wid = s * NC + c
    base = wid * BPW

    pltpu.sync_copy(uidx_hbm.at[pl.ds(base, BPW)], uidx_v)
    pltpu.sync_copy(pidx_hbm.at[pl.ds(base, BPW)], pidx_v)

    pltpu.async_copy(ubias_hbm.at[uidx_v], ub_v, sem2)
    pltpu.async_copy(pbias_hbm.at[pidx_v], pb_v, sem2)

    last_lane = lax.iota(jnp.int32, L) == (L - 1)

    def chunk(cc, carry):
        cbase = cc * CH

        def issue(g, carry):
            iu = uidx_v[pl.ds(cbase + g * L, L)]
            ip = pidx_v[pl.ds(cbase + g * L, L)]
            ju = lax.shift_right_logical(iu, 1)
            jp = lax.shift_right_logical(ip, 1)
            for i in range(L):
                rl = g * L + i
                pltpu.async_copy(uq_hbm.at[pl.ds(ju[i], 1)],
                                 urows_v.at[pl.ds(rl, 1)], sem)
                pltpu.async_copy(pq_hbm.at[pl.ds(jp[i], 1)],
                                 prows_v.at[pl.ds(rl, 1)], sem)
            return carry

        lax.fori_loop(0, CHG, issue, 0)
        pltpu.make_async_copy(uq_hbm.at[pl.ds(0, CH)], urows_v, sem).wait()
        pltpu.make_async_copy(pq_hbm.at[pl.ds(0, CH)], prows_v, sem).wait()

        def body(g, carry):
            iu = uidx_v[pl.ds(cbase + g * L, L)]
            ip = pidx_v[pl.ds(cbase + g * L, L)]
            qu = (iu & 1) * D
            qp = (ip & 1) * D
            for i in range(L):
                rl = g * L + i
                acc = None
                for k in range(D // L):
                    uw = urows_v[rl, pl.ds(qu[i] + k * L, L)]
                    pw = prows_v[rl, pl.ds(qp[i] + k * L, L)]
                    t = uw * pw
                    acc = t if acc is None else acc + t
                tot = plsc.cumsum(acc)
                plsc.store_scatter(out_v,
                                   [jnp.full((L,), cbase + rl, jnp.int32)],
                                   tot, mask=last_lane)
            return carry

        lax.fori_loop(0, CHG, body, 0)
        return carry

    lax.fori_loop(0, NCH, chunk, 0)

    pltpu.make_async_copy(ubias_hbm.at[pl.ds(0, BPW)], ub_v, sem2).wait()
    pltpu.make_async_copy(pbias_hbm.at[pl.ds(0, BPW)], pb_v, sem2).wait()

    def bias_body(g, carry):
        sl = pl.ds(g * L, L)
        out_v[sl] = out_v[sl] + ub_v[sl] + pb_v[sl]
        return carry

    lax.fori_loop(0, BPW // L, bias_body, 0)
    pltpu.sync_copy(out_v, out_hbm.at[pl.ds(base, BPW)])


@jax.jit
def _run(uidx, pidx, uq, pq, ubias, pbias):
    mesh = plsc.VectorSubcoreMesh(core_axis_name="c", subcore_axis_name="s")
    kern = functools.partial(
        pl.kernel,
        out_type=jax.ShapeDtypeStruct((B,), jnp.float32),
        mesh=mesh,
        compiler_params=pltpu.CompilerParams(needs_layout_passes=False),
        scratch_types=[
            pltpu.VMEM((BPW,), jnp.int32),
            pltpu.VMEM((BPW,), jnp.int32),
            pltpu.VMEM((CH, PW), jnp.float32),
            pltpu.VMEM((CH, PW), jnp.float32),
            pltpu.VMEM((BPW,), jnp.float32),
            pltpu.VMEM((BPW,), jnp.float32),
            pltpu.VMEM((BPW,), jnp.float32),
            pltpu.SemaphoreType.DMA,
            pltpu.SemaphoreType.DMA,
        ],
    )(_sc_body)
    return kern(uidx, pidx, uq, pq, ubias, pbias)


def _pack(emb):
    return emb.reshape(N // 2, 2 * D)


def kernel(inputs, user_emb, place_emb, user_bias, place_bias):
    uidx = inputs[:, 0]
    pidx = inputs[:, 1]
    out = _run(uidx, pidx, _pack(user_emb), _pack(place_emb),
               user_bias.reshape(-1), place_bias.reshape(-1))
    return out.reshape(B, 1)


# speedup vs baseline: 9.1677x; 1.0031x over previous
import functools

import jax
import jax.numpy as jnp
from jax import lax
from jax.experimental import pallas as pl
from jax.experimental.pallas import tpu as pltpu
from jax.experimental.pallas import tpu_sc as plsc

B = 16384
D = 64
NC = 2
NS = 16
NW = NC * NS
BPW = B // NW
L = 16


def _sc_body(uidx_hbm, pidx_hbm, uemb_hbm, pemb_hbm, ubias_hbm, pbias_hbm,
             out_hbm, uidx_v, pidx_v, urows_v, prows_v, ub_v, pb_v, out_v,
             sem):
    c = lax.axis_index("c")
    s = lax.axis_index("s")
    wid = s * NC + c
    base = wid * BPW

    pltpu.sync_copy(uidx_hbm.at[pl.ds(base, BPW)], uidx_v)
    pltpu.sync_copy(pidx_hbm.at[pl.ds(base, BPW)], pidx_v)

    cp1 = pltpu.async_copy(uemb_hbm.at[uidx_v], urows_v, sem)
    cp2 = pltpu.async_copy(pemb_hbm.at[pidx_v], prows_v, sem)
    cp3 = pltpu.async_copy(ubias_hbm.at[uidx_v], ub_v, sem)
    cp4 = pltpu.async_copy(pbias_hbm.at[pidx_v], pb_v, sem)
    cp1.wait()
    cp2.wait()
    cp3.wait()
    cp4.wait()

    last_lane = lax.iota(jnp.int32, L) == (L - 1)

    def body(r, carry):
        acc = urows_v[r, pl.ds(0, L)] * prows_v[r, pl.ds(0, L)]
        for k in range(1, D // L):
            acc = acc + urows_v[r, pl.ds(L * k, L)] * prows_v[r, pl.ds(L * k, L)]
        tot = plsc.cumsum(acc)
        plsc.store_scatter(out_v, [jnp.full((L,), r, jnp.int32)], tot,
                           mask=last_lane)
        return carry

    lax.fori_loop(0, BPW, body, 0)

    def bias_body(g, carry):
        sl = pl.ds(g * L, L)
        out_v[sl] = out_v[sl] + ub_v[sl] + pb_v[sl]
        return carry

    lax.fori_loop(0, BPW // L, bias_body, 0)
    pltpu.sync_copy(out_v, out_hbm.at[pl.ds(base, BPW)])


@jax.jit
def _run(uidx, pidx, user_emb, place_emb, user_bias, place_bias):
    mesh = plsc.VectorSubcoreMesh(core_axis_name="c", subcore_axis_name="s")
    kern = functools.partial(
        pl.kernel,
        out_type=jax.ShapeDtypeStruct((B,), jnp.float32),
        mesh=mesh,
        compiler_params=pltpu.CompilerParams(
            needs_layout_passes=False, use_tc_tiling_on_sc=False),
        scratch_types=[
            pltpu.VMEM((BPW,), jnp.int32),
            pltpu.VMEM((BPW,), jnp.int32),
            pltpu.VMEM((BPW, D), jnp.float32),
            pltpu.VMEM((BPW, D), jnp.float32),
            pltpu.VMEM((BPW,), jnp.float32),
            pltpu.VMEM((BPW,), jnp.float32),
            pltpu.VMEM((BPW,), jnp.float32),
            pltpu.SemaphoreType.DMA,
        ],
    )(_sc_body)
    return kern(uidx, pidx, user_emb, place_emb, user_bias, place_bias)


def kernel(inputs, user_emb, place_emb, user_bias, place_bias):
    uidx = inputs[:, 0]
    pidx = inputs[:, 1]
    out = _run(uidx, pidx, user_emb, place_emb,
               user_bias.reshape(-1), place_bias.reshape(-1))
    return out.reshape(B, 1)
